# baseline (speedup 1.0000x reference)
import functools

import jax
import jax.numpy as jnp
from jax.experimental import pallas as pl
from jax.experimental.pallas import tpu as pltpu

DIM = 4096
BZ = 4
SEQ = 4096
NUM_EXPERTS = 64
R = 8

ROWS = 256
NBUF = 4
NC = BZ * SEQ // ROWS
CPB = SEQ // ROWS
OUT_LANES = 128


def _router_kernel(x_hbm, wat_ref, wbt_ref, outa_ref, outb_ref,
                   buf_ref, means_ref, sems):

    def chunk_copy(i, slot):
        return pltpu.make_async_copy(
            x_hbm.at[pl.ds(i * ROWS, ROWS), :], buf_ref.at[slot],
            sems.at[slot])

    for j in range(NBUF):
        chunk_copy(j, j).start()

    def step(i0, acc):
        for j in range(NBUF):
            i = i0 * NBUF + j
            chunk_copy(i, j).wait()
            for k in range(ROWS // 8):
                acc = acc + buf_ref[j, 8 * k:8 * k + 8, :]

            @pl.when(i0 < NC // NBUF - 1)
            def _():
                chunk_copy(i + NBUF, j).start()

            is_last = jax.lax.rem(i, CPB) == CPB - 1

            @pl.when(is_last)
            def _():
                s4 = acc[0:4, :] + acc[4:8, :]
                s2 = s4[0:2, :] + s4[2:4, :]
                s1 = s2[0:1, :] + s2[1:2, :]
                mean = s1 * (1.0 / SEQ)
                b = i // CPB
                for bb in range(BZ):
                    @pl.when(b == bb)
                    def _():
                        means_ref[bb:bb + 1, :] = mean

            acc = jnp.where(is_last, jnp.zeros_like(acc), acc)
        return acc

    jax.lax.fori_loop(0, NC // NBUF, step, jnp.zeros((8, DIM), jnp.float32))

    means = means_ref[0:BZ, :]

    def topk_rows(logits):
        lanes = jax.lax.broadcasted_iota(jnp.int32, (1, NUM_EXPERTS), 1)
        out_lanes = jax.lax.broadcasted_iota(jnp.int32, (1, OUT_LANES), 1)
        vals = logits
        rows = jnp.zeros((BZ, OUT_LANES), dtype=jnp.int32)
        for i in range(R):
            m = jnp.max(vals, axis=1, keepdims=True)
            cand = jnp.where(vals == m, lanes, NUM_EXPERTS)
            idx = jnp.min(cand, axis=1, keepdims=True)
            rows = jnp.where(out_lanes == i, idx, rows)
            vals = jnp.where(lanes == idx, -jnp.inf, vals)
        return rows

    logits_a = jax.lax.dot_general(
        means, wat_ref[...],
        dimension_numbers=(((1,), (0,)), ((), ())),
        preferred_element_type=jnp.float32,
    )
    logits_b = jax.lax.dot_general(
        means, wbt_ref[...],
        dimension_numbers=(((1,), (0,)), ((), ())),
        preferred_element_type=jnp.float32,
    )
    outa_ref[...] = topk_rows(logits_a).reshape(BZ, 1, OUT_LANES)
    outb_ref[...] = topk_rows(logits_b).reshape(BZ, 1, OUT_LANES)


@functools.partial(jax.jit, static_argnames=("interpret",))
def kernel(input_x, WA, WB, interpret=False):
    xr = input_x.reshape(BZ * SEQ, DIM)
    wat = WA.T
    wbt = WB.T

    out_shape = jax.ShapeDtypeStruct((BZ, 1, OUT_LANES), jnp.int32)
    outa, outb = pl.pallas_call(
        _router_kernel,
        in_specs=[
            pl.BlockSpec(memory_space=pltpu.MemorySpace.HBM),
            pl.BlockSpec(memory_space=pltpu.MemorySpace.VMEM),
            pl.BlockSpec(memory_space=pltpu.MemorySpace.VMEM),
        ],
        out_specs=[
            pl.BlockSpec(memory_space=pltpu.MemorySpace.VMEM),
            pl.BlockSpec(memory_space=pltpu.MemorySpace.VMEM),
        ],
        out_shape=[out_shape, out_shape],
        scratch_shapes=[
            pltpu.VMEM((NBUF, ROWS, DIM), jnp.float32),
            pltpu.VMEM((8, DIM), jnp.float32),
            pltpu.SemaphoreType.DMA((NBUF,)),
        ],
        interpret=interpret,
    )(xr, wat, wbt)

    return (outa[:, 0, :R], outb[:, 0, :R])


# speedup vs baseline: 1.0022x; 1.0022x over previous
import functools

import jax
import jax.numpy as jnp
from jax.experimental import pallas as pl
from jax.experimental.pallas import tpu as pltpu

DIM = 4096
BZ = 4
SEQ = 4096
NUM_EXPERTS = 64
R = 8
OUT_LANES = 128

SLOT_ROWS = 1024
NSLOT = 3

_RAMP = [64, 64, 64, 64, 128, 128, 256, 256, 512, 512, 1024, 1024]
_CHUNKS = []
for _b in range(BZ):
    _sizes = _RAMP if _b == 0 else [SLOT_ROWS] * (SEQ // SLOT_ROWS)
    _r0 = _b * SEQ
    for _s in _sizes:
        _CHUNKS.append((_r0, _s, _b, _r0 + _s == (_b + 1) * SEQ))
        _r0 += _s
NC = len(_CHUNKS)


def _router_kernel(x_hbm, w_ref, outa_ref, outb_ref,
                   buf_ref, lg_ref, sems):

    def chunk_copy(i):
        row0, rows, _, _ = _CHUNKS[i]
        slot = i % NSLOT
        return pltpu.make_async_copy(
            x_hbm.at[pl.ds(row0, rows), :],
            buf_ref.at[slot, pl.ds(0, rows), :],
            sems.at[slot])

    def gate_logits(acc):
        s4 = acc[0:4, :] + acc[4:8, :]
        s2 = s4[0:2, :] + s4[2:4, :]
        s1 = s2[0:1, :] + s2[1:2, :]
        mean = s1 * (1.0 / SEQ)
        return jax.lax.dot_general(
            mean, w_ref[...],
            dimension_numbers=(((1,), (0,)), ((), ())),
            preferred_element_type=jnp.float32,
        )

    for i in range(NSLOT):
        chunk_copy(i).start()

    acc = None
    for i, (row0, rows, b, is_last) in enumerate(_CHUNKS):
        slot = i % NSLOT
        chunk_copy(i).wait()
        for k in range(rows // 8):
            g = buf_ref[slot, 8 * k:8 * k + 8, :]
            acc = g if acc is None else acc + g
        if i + NSLOT < NC:
            chunk_copy(i + NSLOT).start()
        if is_last:
            lg_ref[b:b + 1, :] = gate_logits(acc)
            acc = None

    lg = lg_ref[0:BZ, :]

    def topk_rows(vals):
        lanes = jax.lax.broadcasted_iota(jnp.int32, (1, NUM_EXPERTS), 1)
        out_lanes = jax.lax.broadcasted_iota(jnp.int32, (1, OUT_LANES), 1)
        rows = jnp.zeros((BZ, OUT_LANES), dtype=jnp.int32)
        for i in range(R):
            m = jnp.max(vals, axis=1, keepdims=True)
            cand = jnp.where(vals == m, lanes, NUM_EXPERTS)
            idx = jnp.min(cand, axis=1, keepdims=True)
            rows = jnp.where(out_lanes == i, idx, rows)
            vals = jnp.where(lanes == idx, -jnp.inf, vals)
        return rows

    outa_ref[...] = topk_rows(lg[:, 0:NUM_EXPERTS]).reshape(BZ, 1, OUT_LANES)
    outb_ref[...] = topk_rows(lg[:, NUM_EXPERTS:]).reshape(BZ, 1, OUT_LANES)


@functools.partial(jax.jit, static_argnames=("interpret",))
def kernel(input_x, WA, WB, interpret=False):
    xr = input_x.reshape(BZ * SEQ, DIM)
    w = jnp.concatenate([WA.T, WB.T], axis=1)

    out_shape = jax.ShapeDtypeStruct((BZ, 1, OUT_LANES), jnp.int32)
    outa, outb = pl.pallas_call(
        _router_kernel,
        in_specs=[
            pl.BlockSpec(memory_space=pltpu.MemorySpace.HBM),
            pl.BlockSpec(memory_space=pltpu.MemorySpace.VMEM),
        ],
        out_specs=[
            pl.BlockSpec(memory_space=pltpu.MemorySpace.VMEM),
            pl.BlockSpec(memory_space=pltpu.MemorySpace.VMEM),
        ],
        out_shape=[out_shape, out_shape],
        scratch_shapes=[
            pltpu.VMEM((NSLOT, SLOT_ROWS, DIM), jnp.float32),
            pltpu.VMEM((8, 2 * NUM_EXPERTS), jnp.float32),
            pltpu.SemaphoreType.DMA((NSLOT,)),
        ],
        interpret=interpret,
    )(xr, w)

    return (outa[:, 0, :R], outb[:, 0, :R])


# speedup vs baseline: 1.0494x; 1.0471x over previous
import functools

import jax
import jax.numpy as jnp
from jax.experimental import pallas as pl
from jax.experimental.pallas import tpu as pltpu

DIM = 4096
BZ = 4
SEQ = 4096
NUM_EXPERTS = 64
R = 8
OUT_LANES = 128

SLOT_ROWS = 1024
NSLOT = 3

_RAMP = [64, 64, 64, 64, 128, 128, 256, 256, 512, 512, 1024, 1024]
_CHUNKS = []
for _b in range(BZ):
    _sizes = _RAMP if _b == 0 else [SLOT_ROWS] * (SEQ // SLOT_ROWS)
    _r0 = 0
    for _s in _sizes:
        _CHUNKS.append((_b, _r0, _s, _r0 + _s == SEQ))
        _r0 += _s
NC = len(_CHUNKS)


def _router_kernel(x_hbm, wa_ref, wb_ref, outa_ref, outb_ref,
                   buf_ref, lg_ref, sems):

    def chunk_copy(i):
        b, r0, rows, _ = _CHUNKS[i]
        slot = i % NSLOT
        return pltpu.make_async_copy(
            x_hbm.at[b, pl.ds(r0, rows), :],
            buf_ref.at[slot, pl.ds(0, rows), :],
            sems.at[slot])

    def gate_logits(acc):
        s4 = acc[0:4, :] + acc[4:8, :]
        s2 = s4[0:2, :] + s4[2:4, :]
        s1 = s2[0:1, :] + s2[1:2, :]
        mean = s1 * (1.0 / SEQ)
        la = jax.lax.dot_general(
            mean, wa_ref[...],
            dimension_numbers=(((1,), (1,)), ((), ())),
            preferred_element_type=jnp.float32,
        )
        lb = jax.lax.dot_general(
            mean, wb_ref[...],
            dimension_numbers=(((1,), (1,)), ((), ())),
            preferred_element_type=jnp.float32,
        )
        return la, lb

    for i in range(NSLOT):
        chunk_copy(i).start()

    acc = None
    for i, (b, r0, rows, is_last) in enumerate(_CHUNKS):
        slot = i % NSLOT
        chunk_copy(i).wait()
        for k in range(rows // 8):
            g = buf_ref[slot, 8 * k:8 * k + 8, :]
            acc = g if acc is None else acc + g
        if i + NSLOT < NC:
            chunk_copy(i + NSLOT).start()
        if is_last:
            la, lb = gate_logits(acc)
            lg_ref[b:b + 1, 0:NUM_EXPERTS] = la
            lg_ref[b:b + 1, NUM_EXPERTS:] = lb
            acc = None

    lg = lg_ref[0:BZ, :]

    def topk_rows(vals):
        lanes = jax.lax.broadcasted_iota(jnp.int32, (1, NUM_EXPERTS), 1)
        out_lanes = jax.lax.broadcasted_iota(jnp.int32, (1, OUT_LANES), 1)
        rows = jnp.zeros((BZ, OUT_LANES), dtype=jnp.int32)
        for i in range(R):
            m = jnp.max(vals, axis=1, keepdims=True)
            cand = jnp.where(vals == m, lanes, NUM_EXPERTS)
            idx = jnp.min(cand, axis=1, keepdims=True)
            rows = jnp.where(out_lanes == i, idx, rows)
            vals = jnp.where(lanes == idx, -jnp.inf, vals)
        return rows

    outa_ref[...] = topk_rows(lg[:, 0:NUM_EXPERTS])[:, 0:R]
    outb_ref[...] = topk_rows(lg[:, NUM_EXPERTS:])[:, 0:R]


@functools.partial(jax.jit, static_argnames=("interpret",))
def kernel(input_x, WA, WB, interpret=False):
    out_shape = jax.ShapeDtypeStruct((BZ, R), jnp.int32)
    return pl.pallas_call(
        _router_kernel,
        in_specs=[
            pl.BlockSpec(memory_space=pltpu.MemorySpace.HBM),
            pl.BlockSpec(memory_space=pltpu.MemorySpace.VMEM),
            pl.BlockSpec(memory_space=pltpu.MemorySpace.VMEM),
        ],
        out_specs=[
            pl.BlockSpec(memory_space=pltpu.MemorySpace.VMEM),
            pl.BlockSpec(memory_space=pltpu.MemorySpace.VMEM),
        ],
        out_shape=[out_shape, out_shape],
        scratch_shapes=[
            pltpu.VMEM((NSLOT, SLOT_ROWS, DIM), jnp.float32),
            pltpu.VMEM((8, 2 * NUM_EXPERTS), jnp.float32),
            pltpu.SemaphoreType.DMA((NSLOT,)),
        ],
        interpret=interpret,
    )(input_x, WA, WB)
